# baseline (speedup 1.0000x reference)
import functools

import jax
import jax.numpy as jnp
from jax import lax
from jax.experimental import pallas as pl
from jax.experimental.pallas import tpu as pltpu

N = 10000
E = 320000
F = 128
D = 16
L = 3
G = 64
DN = 256


def _prep_body(h_ref, wd_ref, ws_ref, bd_ref, td_ref, ts_ref):
    h = h_ref[...]
    td_ref[...] = jnp.dot(h, wd_ref[...], preferred_element_type=jnp.float32) + bd_ref[...]
    ts_ref[...] = jnp.dot(h, ws_ref[...], preferred_element_type=jnp.float32)


def _prep(h, wd, ws, bd):
    blk = 2000
    return pl.pallas_call(
        _prep_body,
        grid=(N // blk,),
        in_specs=[
            pl.BlockSpec((blk, F), lambda i: (i, 0)),
            pl.BlockSpec((F, 2 * F), lambda i: (0, 0)),
            pl.BlockSpec((F, 2 * F), lambda i: (0, 0)),
            pl.BlockSpec((1, 2 * F), lambda i: (0, 0)),
        ],
        out_specs=[
            pl.BlockSpec((blk, 2 * F), lambda i: (i, 0)),
            pl.BlockSpec((blk, 2 * F), lambda i: (i, 0)),
        ],
        out_shape=[
            jax.ShapeDtypeStruct((N, 2 * F), jnp.float32),
            jax.ShapeDtypeStruct((N, 2 * F), jnp.float32),
        ],
    )(h, wd, ws, bd)


def _eproj_body(e_ref, w_ref, r_ref):
    r_ref[0] = jnp.dot(e_ref[...], w_ref[0], preferred_element_type=jnp.float32)


def _eproj(edge_attr, we):
    blk = 16000
    return pl.pallas_call(
        _eproj_body,
        grid=(L, E // blk),
        in_specs=[
            pl.BlockSpec((blk, D), lambda l, i: (i, 0)),
            pl.BlockSpec((1, D, 2 * F), lambda l, i: (l, 0, 0)),
        ],
        out_specs=pl.BlockSpec((1, blk, 2 * F), lambda l, i: (l, i, 0)),
        out_shape=jax.ShapeDtypeStruct((L, E, 2 * F), jnp.float32),
    )(edge_attr, we)


def _post_body(p_ref, h_ref, g_ref, b_ref, o_ref):
    agg = p_ref[0] + p_ref[1]
    mu = jnp.mean(agg, axis=0, keepdims=True)
    cent = agg - mu
    var = jnp.mean(cent * cent, axis=0, keepdims=True)
    scale = g_ref[...] * lax.rsqrt(var + 1e-5)
    o_ref[...] = cent * scale + b_ref[...] + h_ref[...]


def _post(partials, h, gamma, beta):
    return pl.pallas_call(
        _post_body,
        out_shape=jax.ShapeDtypeStruct((N, F), jnp.float32),
    )(partials, h, gamma.reshape(1, F), beta.reshape(1, F))


def _pool_head_body(h_ref, batch_ref, w1_ref, b1_ref, g2_ref, be2_ref,
                    w2_ref, b2_ref, o_ref):
    h = h_ref[...]
    b = batch_ref[...]
    neg = jnp.float32(-jnp.inf)
    rows = []
    for g in range(G):
        m = (b == g)
        rows.append(jnp.max(jnp.where(m, h, neg), axis=0))
    pooled = jnp.stack(rows)
    d = jnp.dot(pooled, w1_ref[...], preferred_element_type=jnp.float32) + b1_ref[...]
    d = jnp.maximum(d, 0.0)
    mu = jnp.mean(d, axis=0, keepdims=True)
    cent = d - mu
    var = jnp.mean(cent * cent, axis=0, keepdims=True)
    d = cent * (g2_ref[...] * lax.rsqrt(var + 1e-5)) + be2_ref[...]
    out = jnp.dot(d, w2_ref[...], preferred_element_type=jnp.float32) + b2_ref[...]
    o_ref[...] = jax.nn.sigmoid(out)


def _pool_head(h, batch, W1, b1, g2, be2, W2, b2):
    return pl.pallas_call(
        _pool_head_body,
        out_shape=jax.ShapeDtypeStruct((G, 1), jnp.float32),
    )(h, batch.reshape(N, 1), W1, b1.reshape(1, DN), g2.reshape(1, DN),
      be2.reshape(1, DN), W2, b2.reshape(1, 1))


def _edge_stage(td, ts, r, src, dst):
    gd = td[dst]
    gs = ts[src]
    gf = gd[:, :F] + gs[:, :F] + r[:, :F]
    gv = gd[:, F:] + gs[:, F:] + r[:, F:]
    m = jax.nn.sigmoid(gf) * jax.nn.softplus(gv)
    half = E // 2
    p0 = jax.ops.segment_sum(m[:half], dst[:half], num_segments=N)
    p1 = jax.ops.segment_sum(m[half:], dst[half:], num_segments=N)
    return jnp.stack([p0, p1])


def kernel(x, edge_attr, Wf, bf, Ws, bs, gamma, beta, W1, b1, g2, be2, W2, b2,
           edge_index, batch):
    src = edge_index[0]
    dst = edge_index[1]

    wd = jnp.concatenate([Wf[:, :F, :], Ws[:, :F, :]], axis=2)
    wsrc = jnp.concatenate([Wf[:, F:2 * F, :], Ws[:, F:2 * F, :]], axis=2)
    we = jnp.concatenate([Wf[:, 2 * F:, :], Ws[:, 2 * F:, :]], axis=2)
    bd = jnp.concatenate([bf, bs], axis=1)

    r_all = _eproj(edge_attr, we)

    h = x
    for l in range(L):
        td, tsx = _prep(h, wd[l], wsrc[l], bd[l].reshape(1, 2 * F))
        partials = _edge_stage(td, tsx, r_all[l], src, dst)
        h = _post(partials, h, gamma[l], beta[l])

    return _pool_head(h, batch, W1, b1, g2, be2, W2, b2)
